# baseline (speedup 1.0000x reference)
import functools

import jax
import jax.numpy as jnp
from jax.experimental import pallas as pl

_NPOINT = 1024
_RADIUS = [0.1, 0.2, 0.4]
_NSAMPLE = [16, 32, 64]
_MLP_DIMS = [[32, 32, 64], [64, 64, 128], [64, 96, 128]]
_KSIZES = [1, 3, 5]
_BN_EPS = 1e-5


def _fps(xyz, npoint):
    B, N, _ = xyz.shape
    centroids0 = jnp.zeros((B, npoint), dtype=jnp.int32)
    distance0 = jnp.full((B, N), 1e10, dtype=jnp.float32)
    farthest0 = jnp.zeros((B,), dtype=jnp.int32)

    def body(i, state):
        centroids, distance, farthest = state
        centroids = centroids.at[:, i].set(farthest)
        centroid = jnp.take_along_axis(
            xyz, farthest[:, None, None].astype(jnp.int32), axis=1)
        dist = jnp.sum((xyz - centroid) ** 2, axis=-1)
        distance = jnp.minimum(distance, dist)
        farthest = jnp.argmax(distance, axis=-1).astype(jnp.int32)
        return (centroids, distance, farthest)

    centroids, _, _ = jax.lax.fori_loop(
        0, npoint, body, (centroids0, distance0, farthest0))
    return centroids


def _sqdist(a, b):
    return (jnp.sum(a ** 2, axis=-1)[:, :, None]
            + jnp.sum(b ** 2, axis=-1)[:, None, :]
            - 2.0 * jnp.einsum('bsc,bnc->bsn', a, b))


def _ball_query(radius, nsample, xyz, new_xyz):
    B, N, _ = xyz.shape
    sqr = _sqdist(new_xyz, xyz)
    gidx = jnp.broadcast_to(jnp.arange(N, dtype=jnp.int32),
                            (B, new_xyz.shape[1], N))
    gidx = jnp.where(sqr > radius ** 2, N, gidx)
    gidx = jnp.sort(gidx, axis=-1)[:, :, :nsample]
    first = gidx[:, :, :1]
    gidx = jnp.where(gidx == N, jnp.broadcast_to(first, gidx.shape), gidx)
    return gidx


def _gather(p, idx):
    return jax.vmap(lambda pb, ib: pb[ib])(p, idx)


def _layer_first_body(x_ref, w_ref, b_ref, out_ref, part_ref):
    x = x_ref[...]
    g = jnp.dot(x, w_ref[...], preferred_element_type=jnp.float32) + b_ref[...]
    out_ref[...] = g

    @pl.when(pl.program_id(0) == 0)
    def _():
        part_ref[...] = jnp.zeros_like(part_ref)

    part_ref[0:1, :] += jnp.sum(g, axis=0, keepdims=True)
    part_ref[1:2, :] += jnp.sum(g * g, axis=0, keepdims=True)


def _layer_body(x_ref, sc_ref, sh_ref, w_ref, b_ref, out_ref, part_ref):
    x = x_ref[...]
    x = jnp.maximum(x * sc_ref[...] + sh_ref[...], 0.0)
    g = jnp.dot(x, w_ref[...], preferred_element_type=jnp.float32) + b_ref[...]
    out_ref[...] = g

    @pl.when(pl.program_id(0) == 0)
    def _():
        part_ref[...] = jnp.zeros_like(part_ref)

    part_ref[0:1, :] += jnp.sum(g, axis=0, keepdims=True)
    part_ref[1:2, :] += jnp.sum(g * g, axis=0, keepdims=True)


def _run_layer(x, w, b, scale=None, shift=None, tile=1024):
    M, cin = x.shape
    cout = w.shape[1]
    grid = M // tile
    b2 = b[None, :]
    outs = [jax.ShapeDtypeStruct((M, cout), jnp.float32),
            jax.ShapeDtypeStruct((8, cout), jnp.float32)]
    out_specs = [pl.BlockSpec((tile, cout), lambda i: (i, 0)),
                 pl.BlockSpec((8, cout), lambda i: (0, 0))]
    wspec = pl.BlockSpec((cin, cout), lambda i: (0, 0))
    bspec = pl.BlockSpec((1, cout), lambda i: (0, 0))
    xspec = pl.BlockSpec((tile, cin), lambda i: (i, 0))
    if scale is None:
        g, part = pl.pallas_call(
            _layer_first_body,
            grid=(grid,),
            in_specs=[xspec, wspec, bspec],
            out_specs=out_specs,
            out_shape=outs,
        )(x, w, b2)
    else:
        sspec = pl.BlockSpec((1, cin), lambda i: (0, 0))
        g, part = pl.pallas_call(
            _layer_body,
            grid=(grid,),
            in_specs=[xspec, sspec, sspec, wspec, bspec],
            out_specs=out_specs,
            out_shape=outs,
        )(x, scale[None, :], shift[None, :], w, b2)
    return g, part


def _bn_coeffs(part, M, gamma, beta):
    s, s2 = part[0], part[1]
    mean = s / M
    var = s2 / M - mean * mean
    scale = gamma * jax.lax.rsqrt(var + _BN_EPS)
    shift = beta - mean * scale
    return scale, shift


def _run_pool(g, scale, shift, tile=128):
    BS, K, C = g.shape
    grid = BS // tile
    out = pl.pallas_call(
        _pool_relu_body,
        grid=(grid,),
        in_specs=[pl.BlockSpec((tile, K, C), lambda i: (i, 0, 0)),
                  pl.BlockSpec((1, C), lambda i: (0, 0)),
                  pl.BlockSpec((1, C), lambda i: (0, 0))],
        out_specs=pl.BlockSpec((tile, C), lambda i: (i, 0)),
        out_shape=jax.ShapeDtypeStruct((BS, C), jnp.float32),
    )(g, scale[None, :], shift[None, :])
    return out


def _pool_relu_body(g_ref, sc_ref, sh_ref, out_ref):
    g = g_ref[...]
    sc = sc_ref[...][None]
    sh = sh_ref[...][None]
    x = jnp.maximum(g * sc + sh, 0.0)
    out_ref[...] = jnp.max(x, axis=1)


def _cbam_body(x_ref, w1a_ref, w2a_ref, w1m_ref, w2m_ref, wsp_ref, out_ref):
    x = x_ref[0]
    a = jnp.mean(x, axis=0, keepdims=True)
    m = jnp.max(x, axis=0, keepdims=True)

    def pyramid(v, w1_ref, w2_ref):
        acc = jnp.zeros_like(v)
        for k in range(3):
            h = jnp.maximum(
                jnp.dot(v, w1_ref[k], preferred_element_type=jnp.float32), 0.0)
            acc = acc + jnp.dot(h, w2_ref[k],
                                preferred_element_type=jnp.float32)
        return acc

    logit = pyramid(a, w1a_ref, w2a_ref) + pyramid(m, w1m_ref, w2m_ref)
    ca = jax.nn.sigmoid(logit)
    x_ca = x * ca

    mx = jnp.max(x_ca, axis=1, keepdims=True)
    av = jnp.mean(x_ca, axis=1, keepdims=True)
    mn = jnp.min(x_ca, axis=1, keepdims=True)
    spa = jnp.concatenate([mx, av, mn], axis=1)
    w = wsp_ref[...]
    zero = jnp.zeros((1, 3), jnp.float32)
    prev = jnp.concatenate([zero, spa[:-1]], axis=0)
    nxt = jnp.concatenate([spa[1:], zero], axis=0)
    logit_s = (jnp.sum(prev * w[:, 0][None, :], axis=1, keepdims=True)
               + jnp.sum(spa * w[:, 1][None, :], axis=1, keepdims=True)
               + jnp.sum(nxt * w[:, 2][None, :], axis=1, keepdims=True))
    sa = jax.nn.sigmoid(logit_s)
    out_ref[0] = x_ca * sa


def _run_cbam(x_bsc, w1a, w2a, w1m, w2m, wsp):
    B, S, C = x_bsc.shape
    out = pl.pallas_call(
        _cbam_body,
        grid=(B,),
        in_specs=[pl.BlockSpec((1, S, C), lambda i: (i, 0, 0)),
                  pl.BlockSpec(w1a.shape, lambda i: (0, 0, 0)),
                  pl.BlockSpec(w2a.shape, lambda i: (0, 0, 0)),
                  pl.BlockSpec(w1m.shape, lambda i: (0, 0, 0)),
                  pl.BlockSpec(w2m.shape, lambda i: (0, 0, 0)),
                  pl.BlockSpec(wsp.shape, lambda i: (0, 0))],
        out_specs=pl.BlockSpec((1, S, C), lambda i: (i, 0, 0)),
        out_shape=jax.ShapeDtypeStruct((B, S, C), jnp.float32),
    )(x_bsc, w1a, w2a, w1m, w2m, wsp)
    return out


def kernel(xyz, points, params):
    B, N, _ = xyz.shape
    S = _NPOINT

    fps_idx = _fps(xyz, S)
    new_xyz = _gather(xyz, fps_idx)

    outs = []
    for i, r in enumerate(_RADIUS):
        K = _NSAMPLE[i]
        idx = _ball_query(r, K, xyz, new_xyz)
        g_xyz = _gather(xyz, idx) - new_xyz[:, :, None, :]
        g_pts = _gather(points, idx)
        grouped = jnp.concatenate([g_xyz, g_pts], axis=-1)
        M = B * S * K
        x = grouped.reshape(M, -1)

        layers = params['branches'][i]
        scale = shift = None
        g = x
        for li, layer in enumerate(layers):
            w = layer['W'].T
            g, part = _run_layer(g, w, layer['b'], scale, shift)
            scale, shift = _bn_coeffs(part, M, layer['gamma'], layer['beta'])
        c3 = g.shape[-1]
        pooled = _run_pool(g.reshape(B * S, K, c3), scale, shift)
        outs.append(pooled.reshape(B, S, c3))

    x_bsc = jnp.concatenate(outs, axis=-1)

    cb = params['cbam']
    def eff(br_list):
        w1 = jnp.stack([br['w1'][:, :, k // 2].T
                        for br, k in zip(br_list, _KSIZES)])
        w2 = jnp.stack([br['w2'][:, :, k // 2].T
                        for br, k in zip(br_list, _KSIZES)])
        return w1, w2
    w1a, w2a = eff(cb['avg'])
    w1m, w2m = eff(cb['max'])
    wsp = cb['w_spatial'][0]

    y = _run_cbam(x_bsc, w1a, w2a, w1m, w2m, wsp)
    x_out = jnp.transpose(y, (0, 2, 1))
    return new_xyz, x_out


# speedup vs baseline: 1.9214x; 1.9214x over previous
import functools

import jax
import jax.numpy as jnp
from jax import lax
from jax.experimental import pallas as pl
from jax.experimental.pallas import tpu as pltpu
from jax.experimental.pallas import tpu_sc as plsc

_NPOINT = 1024
_RADIUS = [0.1, 0.2, 0.4]
_NSAMPLE = [16, 32, 64]
_KSIZES = [1, 3, 5]
_BN_EPS = 1e-5
_CW = 128


def _fps(xyz, npoint):
    B, N, _ = xyz.shape
    centroids0 = jnp.zeros((B, npoint), dtype=jnp.int32)
    distance0 = jnp.full((B, N), 1e10, dtype=jnp.float32)
    farthest0 = jnp.zeros((B,), dtype=jnp.int32)

    def body(i, state):
        centroids, distance, farthest = state
        centroids = centroids.at[:, i].set(farthest)
        centroid = jnp.take_along_axis(
            xyz, farthest[:, None, None].astype(jnp.int32), axis=1)
        dist = jnp.sum((xyz - centroid) ** 2, axis=-1)
        distance = jnp.minimum(distance, dist)
        farthest = jnp.argmax(distance, axis=-1).astype(jnp.int32)
        return (centroids, distance, farthest)

    centroids, _, _ = jax.lax.fori_loop(
        0, npoint, body, (centroids0, distance0, farthest0))
    return centroids


def _sqdist(a, b):
    return (jnp.sum(a ** 2, axis=-1)[:, :, None]
            + jnp.sum(b ** 2, axis=-1)[:, None, :]
            - 2.0 * jnp.einsum('bsc,bnc->bsn', a, b))


def _ball_query(radius, nsample, xyz, new_xyz):
    B, N, _ = xyz.shape
    sqr = _sqdist(new_xyz, xyz)
    gidx = jnp.broadcast_to(jnp.arange(N, dtype=jnp.int32),
                            (B, new_xyz.shape[1], N))
    gidx = jnp.where(sqr > radius ** 2, N, gidx)
    gidx = jnp.sort(gidx, axis=-1)[:, :, :nsample]
    first = gidx[:, :, :1]
    gidx = jnp.where(gidx == N, jnp.broadcast_to(first, gidx.shape), gidx)
    return gidx


def _gather(p, idx):
    return jax.vmap(lambda pb, ib: pb[ib])(p, idx)


def _bcast(v, lane_v):
    return v.at[lane_v].get(mode='promise_in_bounds')


def _prefix16(m):
    ln = lax.iota(jnp.int32, 16)
    v = m
    for k in (1, 2, 4, 8):
        idx = jnp.maximum(ln - k, 0)
        ind = jnp.minimum(jnp.maximum(ln - (k - 1), 0), 1)
        v = v + v.at[idx].get(mode='promise_in_bounds') * ind
    return v


@functools.cache
def _make_sc_group(B, N, S):
    info = plsc.get_sparse_core_info()
    NW = info.num_cores * info.num_subcores
    s_per_w = (B * S) // NW
    n_chunks = N // 16
    K1, K2, K3 = _NSAMPLE
    R1, R2, R3 = [r * r for r in _RADIUS]
    ST1, ST2, ST3 = K1 + 16, K2 + 16, K3 + 16
    mesh = plsc.VectorSubcoreMesh(core_axis_name="c", subcore_axis_name="s")

    @functools.partial(
        pl.kernel,
        out_type=[jax.ShapeDtypeStruct((B * S * K1, _CW), jnp.float32),
                  jax.ShapeDtypeStruct((B * S * K2, _CW), jnp.float32),
                  jax.ShapeDtypeStruct((B * S * K3, _CW), jnp.float32)],
        mesh=mesh,
        scratch_types=[
            pltpu.VMEM((N,), jnp.float32),
            pltpu.VMEM((N,), jnp.float32),
            pltpu.VMEM((N,), jnp.float32),
            pltpu.VMEM((s_per_w,), jnp.float32),
            pltpu.VMEM((s_per_w,), jnp.float32),
            pltpu.VMEM((s_per_w,), jnp.float32),
            pltpu.VMEM((s_per_w * ST1,), jnp.int32),
            pltpu.VMEM((s_per_w * ST2,), jnp.int32),
            pltpu.VMEM((s_per_w * ST3,), jnp.int32),
            pltpu.VMEM((s_per_w * 16,), jnp.int32),
            pltpu.VMEM((s_per_w * 16,), jnp.int32),
            pltpu.VMEM((s_per_w * 16,), jnp.int32),
            pltpu.VMEM((K1,), jnp.int32),
            pltpu.VMEM((K2,), jnp.int32),
            pltpu.VMEM((K3,), jnp.int32),
            pltpu.VMEM((K1, _CW), jnp.float32),
            pltpu.VMEM((K2, _CW), jnp.float32),
            pltpu.VMEM((K3, _CW), jnp.float32),
            pltpu.SemaphoreType.DMA,
        ],
    )
    def group_k(combo_hbm, xs_hbm, ys_hbm, zs_hbm, cx_hbm, cy_hbm, cz_hbm,
                out1, out2, out3, xsr, ysr, zsr, cxr, cyr, czr,
                ib1, ib2, ib3, cb1, cb2, cb3, gi1, gi2, gi3,
                rw1, rw2, rw3, sem):
        wid = lax.axis_index("s") * info.num_cores + lax.axis_index("c")
        base = pl.multiple_of(wid * s_per_w, 8)
        b = wid // (NW // B)
        pt0 = pl.multiple_of(b * N, 8)
        pltpu.sync_copy(xs_hbm.at[pl.ds(pt0, N)], xsr)
        pltpu.sync_copy(ys_hbm.at[pl.ds(pt0, N)], ysr)
        pltpu.sync_copy(zs_hbm.at[pl.ds(pt0, N)], zsr)
        pltpu.sync_copy(cx_hbm.at[pl.ds(base, s_per_w)], cxr)
        pltpu.sync_copy(cy_hbm.at[pl.ds(base, s_per_w)], cyr)
        pltpu.sync_copy(cz_hbm.at[pl.ds(base, s_per_w)], czr)

        lanes = lax.iota(jnp.int32, 16)
        zeros16 = jnp.zeros((16,), jnp.int32)
        lane15 = jnp.full((16,), 15, jnp.int32)
        bn = b * N

        def chunk_step(t, st):
            c1, c2, c3 = st
            j = t // n_chunks
            ch = t - j * n_chunks
            keep = jnp.minimum(ch, 1)
            c1 = c1 * keep
            c2 = c2 * keep
            c3 = c3 * keep
            j16 = pl.multiple_of((j // 16) * 16, 16)
            lane_v = jnp.full((16,), j - j16, jnp.int32)
            cx = _bcast(cxr[pl.ds(j16, 16)], lane_v)
            cy = _bcast(cyr[pl.ds(j16, 16)], lane_v)
            cz = _bcast(czr[pl.ds(j16, 16)], lane_v)
            off = pl.multiple_of(ch * 16, 16)
            xs = xsr[pl.ds(off, 16)]
            ys = ysr[pl.ds(off, 16)]
            zs = zsr[pl.ds(off, 16)]
            dx = xs - cx
            dy = ys - cy
            dz = zs - cz
            d = dx * dx + dy * dy + dz * dz
            ids = lanes + ch * 16
            c16 = pl.multiple_of(j * 16, 16)

            di = plsc.bitcast(d, jnp.int32)

            def push(ibuf, cbuf, cnt, kk, rr, stride):
                import struct as _st
                rbits = _st.unpack('<i', _st.pack('<f', rr))[0]
                mb = jnp.logical_and(di <= rbits, cnt < kk)
                m32 = mb.astype(jnp.int32)
                pref = _prefix16(m32)
                pos = cnt + pref - 1
                pos = pos * m32 + (1 - m32) * (stride - 1) + j * stride
                plsc.store_scatter(ibuf, [pos], ids, mask=mb)
                cnt = cnt + _bcast(pref, lane15)
                cbuf[pl.ds(c16, 16)] = cnt
                return cnt

            c1 = push(ib1, cb1, c1, K1, R1, ST1)
            c2 = push(ib2, cb2, c2, K2, R2, ST2)
            c3 = push(ib3, cb3, c3, K3, R3, ST3)
            return (c1, c2, c3)

        lax.fori_loop(0, s_per_w * n_chunks, chunk_step,
                      (zeros16, zeros16, zeros16))

        def centroid_fin(j, carry):
            def finalize(ibuf, cbuf, gbuf, kk, stride):
                cnt = cbuf[pl.ds(pl.multiple_of(j * 16, 16), 16)]
                b0 = pl.multiple_of(j * stride, 16)
                first = _bcast(ibuf[pl.ds(b0, 16)], zeros16)
                for q in range(kk // 16):
                    v = ibuf[pl.ds(b0 + q * 16, 16)]
                    lp = lanes + q * 16
                    gbuf[q * 16:(q + 1) * 16] = (
                        jnp.where(lp < cnt, v, first) + bn)

            finalize(ib1, cb1, gi1, K1, ST1)
            finalize(ib2, cb2, gi2, K2, ST2)
            finalize(ib3, cb3, gi3, K3, ST3)

            h1 = pltpu.async_copy(combo_hbm.at[gi1], rw1, sem)
            h2 = pltpu.async_copy(combo_hbm.at[gi2], rw2, sem)
            h3 = pltpu.async_copy(combo_hbm.at[gi3], rw3, sem)
            h1.wait()
            h2.wait()
            h3.wait()
            g = base + j
            pltpu.sync_copy(rw1, out1.at[pl.ds(pl.multiple_of(g * K1, 8), K1)])
            pltpu.sync_copy(rw2, out2.at[pl.ds(pl.multiple_of(g * K2, 8), K2)])
            pltpu.sync_copy(rw3, out3.at[pl.ds(pl.multiple_of(g * K3, 8), K3)])
            return carry

        lax.fori_loop(0, s_per_w, centroid_fin, 0)

    return group_k


def _sc_group(combo, xyz, new_xyz):
    B, N, _ = xyz.shape
    S = new_xyz.shape[1]
    xs = xyz[:, :, 0].reshape(B * N)
    ys = xyz[:, :, 1].reshape(B * N)
    zs = xyz[:, :, 2].reshape(B * N)
    cx = new_xyz[:, :, 0].reshape(B * S)
    cy = new_xyz[:, :, 1].reshape(B * S)
    cz = new_xyz[:, :, 2].reshape(B * S)
    return _make_sc_group(B, N, S)(combo, xs, ys, zs, cx, cy, cz)


@functools.cache
def _make_sc_gather(total, width):
    info = plsc.get_sparse_core_info()
    NW = info.num_cores * info.num_subcores
    b_per_w = total // NW
    assert total % (NW * 512) == 0
    n_steps = b_per_w // 512
    mesh = plsc.VectorSubcoreMesh(core_axis_name="c", subcore_axis_name="s")

    @functools.partial(
        pl.kernel,
        out_type=jax.ShapeDtypeStruct((total, width), jnp.float32),
        mesh=mesh,
        scratch_types=[
            pltpu.VMEM((b_per_w // 128, 128), jnp.int32),
            pltpu.VMEM((512, width), jnp.float32),
            pltpu.SemaphoreType.DMA,
        ],
    )
    def gather_k(table_hbm, idx_hbm, out_hbm, idx_v, rows_v, sem):
        wid = lax.axis_index("s") * info.num_cores + lax.axis_index("c")
        base = pl.multiple_of(wid * b_per_w, 512)
        idx_row0 = pl.multiple_of(wid * (b_per_w // 128), 8)
        pltpu.sync_copy(idx_hbm.at[pl.ds(idx_row0, b_per_w // 128)], idx_v)

        def step(j, carry):
            handles = []
            for q in range(4):
                handles.append(pltpu.async_copy(
                    table_hbm.at[idx_v.at[j * 4 + q]],
                    rows_v.at[pl.ds(q * 128, 128)], sem))
            for h in handles:
                h.wait()
            out0 = pl.multiple_of(base + j * 512, 512)
            pltpu.sync_copy(rows_v, out_hbm.at[pl.ds(out0, 512)])
            return carry

        lax.fori_loop(0, n_steps, step, 0)

    return gather_k


def _sc_gather(table, gidx_flat):
    total = gidx_flat.shape[0]
    width = table.shape[-1]
    return _make_sc_gather(total, width)(
        table, gidx_flat.reshape(total // 128, 128))


def _layer_first_body(x_ref, c_ref, w_ref, wx_ref, b_ref, out_ref, part_ref):
    x = x_ref[...]
    g = jnp.dot(x, w_ref[...], preferred_element_type=jnp.float32) + b_ref[...]
    g = g - jnp.dot(c_ref[...], wx_ref[...], preferred_element_type=jnp.float32)
    out_ref[...] = g

    @pl.when(pl.program_id(0) == 0)
    def _():
        part_ref[...] = jnp.zeros_like(part_ref)

    part_ref[0:1, :] += jnp.sum(g, axis=0, keepdims=True)
    part_ref[1:2, :] += jnp.sum(g * g, axis=0, keepdims=True)


def _layer_body(x_ref, sc_ref, sh_ref, w_ref, b_ref, out_ref, part_ref):
    x = x_ref[...]
    x = jnp.maximum(x * sc_ref[...] + sh_ref[...], 0.0)
    g = jnp.dot(x, w_ref[...], preferred_element_type=jnp.float32) + b_ref[...]
    out_ref[...] = g

    @pl.when(pl.program_id(0) == 0)
    def _():
        part_ref[...] = jnp.zeros_like(part_ref)

    part_ref[0:1, :] += jnp.sum(g, axis=0, keepdims=True)
    part_ref[1:2, :] += jnp.sum(g * g, axis=0, keepdims=True)


def _run_layer_first(x, ctr, w, wx, b, tile=1024):
    M, cin = x.shape
    cout = w.shape[1]
    grid = M // tile
    outs = [jax.ShapeDtypeStruct((M, cout), jnp.float32),
            jax.ShapeDtypeStruct((8, cout), jnp.float32)]
    out_specs = [pl.BlockSpec((tile, cout), lambda i: (i, 0)),
                 pl.BlockSpec((8, cout), lambda i: (0, 0))]
    g, part = pl.pallas_call(
        _layer_first_body,
        grid=(grid,),
        in_specs=[pl.BlockSpec((tile, cin), lambda i: (i, 0)),
                  pl.BlockSpec((tile, 3), lambda i: (i, 0)),
                  pl.BlockSpec((cin, cout), lambda i: (0, 0)),
                  pl.BlockSpec((3, cout), lambda i: (0, 0)),
                  pl.BlockSpec((1, cout), lambda i: (0, 0))],
        out_specs=out_specs,
        out_shape=outs,
    )(x, ctr, w, wx, b[None, :])
    return g, part


def _run_layer(x, w, b, scale, shift, tile=1024):
    M, cin = x.shape
    cout = w.shape[1]
    grid = M // tile
    outs = [jax.ShapeDtypeStruct((M, cout), jnp.float32),
            jax.ShapeDtypeStruct((8, cout), jnp.float32)]
    out_specs = [pl.BlockSpec((tile, cout), lambda i: (i, 0)),
                 pl.BlockSpec((8, cout), lambda i: (0, 0))]
    g, part = pl.pallas_call(
        _layer_body,
        grid=(grid,),
        in_specs=[pl.BlockSpec((tile, cin), lambda i: (i, 0)),
                  pl.BlockSpec((1, cin), lambda i: (0, 0)),
                  pl.BlockSpec((1, cin), lambda i: (0, 0)),
                  pl.BlockSpec((cin, cout), lambda i: (0, 0)),
                  pl.BlockSpec((1, cout), lambda i: (0, 0))],
        out_specs=out_specs,
        out_shape=outs,
    )(x, scale[None, :], shift[None, :], w, b[None, :])
    return g, part


def _bn_coeffs(part, M, gamma, beta):
    s, s2 = part[0], part[1]
    mean = s / M
    var = s2 / M - mean * mean
    scale = gamma * jax.lax.rsqrt(var + _BN_EPS)
    shift = beta - mean * scale
    return scale, shift


def _pool_relu_body(g_ref, sc_ref, sh_ref, out_ref):
    g = g_ref[...]
    sc = sc_ref[...][None]
    sh = sh_ref[...][None]
    x = jnp.maximum(g * sc + sh, 0.0)
    out_ref[...] = jnp.max(x, axis=1)


def _run_pool(g, scale, shift, tile=128):
    BS, K, C = g.shape
    grid = BS // tile
    out = pl.pallas_call(
        _pool_relu_body,
        grid=(grid,),
        in_specs=[pl.BlockSpec((tile, K, C), lambda i: (i, 0, 0)),
                  pl.BlockSpec((1, C), lambda i: (0, 0)),
                  pl.BlockSpec((1, C), lambda i: (0, 0))],
        out_specs=pl.BlockSpec((tile, C), lambda i: (i, 0)),
        out_shape=jax.ShapeDtypeStruct((BS, C), jnp.float32),
    )(g, scale[None, :], shift[None, :])
    return out


def _cbam_body(x_ref, w1a_ref, w2a_ref, w1m_ref, w2m_ref, wsp_ref, out_ref):
    x = x_ref[0]
    a = jnp.mean(x, axis=0, keepdims=True)
    m = jnp.max(x, axis=0, keepdims=True)

    def pyramid(v, w1_ref, w2_ref):
        acc = jnp.zeros_like(v)
        for k in range(3):
            h = jnp.maximum(
                jnp.dot(v, w1_ref[k], preferred_element_type=jnp.float32), 0.0)
            acc = acc + jnp.dot(h, w2_ref[k],
                                preferred_element_type=jnp.float32)
        return acc

    logit = pyramid(a, w1a_ref, w2a_ref) + pyramid(m, w1m_ref, w2m_ref)
    ca = jax.nn.sigmoid(logit)
    x_ca = x * ca

    mx = jnp.max(x_ca, axis=1, keepdims=True)
    av = jnp.mean(x_ca, axis=1, keepdims=True)
    mn = jnp.min(x_ca, axis=1, keepdims=True)
    spa = jnp.concatenate([mx, av, mn], axis=1)
    w = wsp_ref[...]
    zero = jnp.zeros((1, 3), jnp.float32)
    prev = jnp.concatenate([zero, spa[:-1]], axis=0)
    nxt = jnp.concatenate([spa[1:], zero], axis=0)
    logit_s = (jnp.sum(prev * w[:, 0][None, :], axis=1, keepdims=True)
               + jnp.sum(spa * w[:, 1][None, :], axis=1, keepdims=True)
               + jnp.sum(nxt * w[:, 2][None, :], axis=1, keepdims=True))
    sa = jax.nn.sigmoid(logit_s)
    out_ref[0] = x_ca * sa


def _run_cbam(x_bsc, w1a, w2a, w1m, w2m, wsp):
    B, S, C = x_bsc.shape
    out = pl.pallas_call(
        _cbam_body,
        grid=(B,),
        in_specs=[pl.BlockSpec((1, S, C), lambda i: (i, 0, 0)),
                  pl.BlockSpec(w1a.shape, lambda i: (0, 0, 0)),
                  pl.BlockSpec(w2a.shape, lambda i: (0, 0, 0)),
                  pl.BlockSpec(w1m.shape, lambda i: (0, 0, 0)),
                  pl.BlockSpec(w2m.shape, lambda i: (0, 0, 0)),
                  pl.BlockSpec(wsp.shape, lambda i: (0, 0))],
        out_specs=pl.BlockSpec((1, S, C), lambda i: (i, 0, 0)),
        out_shape=jax.ShapeDtypeStruct((B, S, C), jnp.float32),
    )(x_bsc, w1a, w2a, w1m, w2m, wsp)
    return out


def kernel(xyz, points, params):
    B, N, _ = xyz.shape
    S = _NPOINT
    CF = points.shape[-1]

    fps_idx = _fps(xyz, S)
    new_xyz = _gather(xyz, fps_idx)

    combo = jnp.concatenate(
        [xyz, points,
         jnp.zeros((B, N, _CW - 3 - CF), jnp.float32)], axis=-1)
    combo = combo.reshape(B * N, _CW)

    new_xyz_flat = new_xyz.reshape(B * S, 3)
    boff = (jnp.arange(B, dtype=jnp.int32) * N)[:, None, None]

    idxs = [_ball_query(r, _NSAMPLE[i], xyz, new_xyz)
            for i, r in enumerate(_RADIUS)]
    idxs = jax.lax.optimization_barrier(idxs)

    outs = []
    prev = None
    for i, r in enumerate(_RADIUS):
        K = _NSAMPLE[i]
        M = B * S * K
        idx = idxs[i]
        if prev is not None:
            idx, _ = jax.lax.optimization_barrier((idx, prev))
        gidx = (idx + boff).reshape(M)
        x = _sc_gather(combo, gidx)
        prev = x
        ctr = jnp.broadcast_to(new_xyz_flat[:, None, :],
                               (B * S, K, 3)).reshape(M, 3)

        layers = params['branches'][i]
        w0 = layers[0]['W']
        wfull = jnp.zeros((_CW, w0.shape[0]), jnp.float32)
        wfull = wfull.at[:3 + CF, :].set(w0.T)
        wx = w0[:, :3].T
        g, part = _run_layer_first(x, ctr, wfull, wx, layers[0]['b'])
        scale, shift = _bn_coeffs(part, M, layers[0]['gamma'],
                                  layers[0]['beta'])
        for layer in layers[1:]:
            g, part = _run_layer(g, layer['W'].T, layer['b'], scale, shift)
            scale, shift = _bn_coeffs(part, M, layer['gamma'], layer['beta'])
        c3 = g.shape[-1]
        pooled = _run_pool(g.reshape(B * S, K, c3), scale, shift)
        outs.append(pooled.reshape(B, S, c3))

    x_bsc = jnp.concatenate(outs, axis=-1)

    cb = params['cbam']
    def eff(br_list):
        w1 = jnp.stack([br['w1'][:, :, k // 2].T
                        for br, k in zip(br_list, _KSIZES)])
        w2 = jnp.stack([br['w2'][:, :, k // 2].T
                        for br, k in zip(br_list, _KSIZES)])
        return w1, w2
    w1a, w2a = eff(cb['avg'])
    w1m, w2m = eff(cb['max'])
    wsp = cb['w_spatial'][0]

    y = _run_cbam(x_bsc, w1a, w2a, w1m, w2m, wsp)
    x_out = jnp.transpose(y, (0, 2, 1))
    return new_xyz, x_out
